# Initial kernel scaffold; baseline (speedup 1.0000x reference)
import jax
import jax.numpy as jnp
from jax.experimental import pallas as pl


def kernel(eb_input, eb_offset, W0, W1, W2, W3):
    raise NotImplementedError("write your pallas kernel here")


# speedup vs baseline: 10.1234x; 10.1234x over previous
import functools

import numpy as np
import jax
import jax.numpy as jnp
from jax import lax
from jax.experimental import pallas as pl
from jax.experimental.pallas import tpu as pltpu
from jax.experimental.pallas import tpu_sc as plsc

NUM_EMB_ROWS = 1_000_000
NUM_IDX = 819_200
NC = 2
NS = 16
NW = NC * NS
LANES = 16

NEP = 1 << 20
IDX_COLS = 128
IDX_ROWS_TOTAL = NUM_IDX // IDX_COLS
ROWS_PER_W = IDX_ROWS_TOTAL // NW
DMA_ROWS = 8
N_DMA = ROWS_PER_W // DMA_ROWS
SP_SLICE = NEP // NS
ZB = 8192

FLAT = 3 * NUM_EMB_ROWS
PER_W = 93_744
CHUNK = 7_680
NCH = 12
TAIL = PER_W - NCH * CHUNK
EXTRA = FLAT - NW * PER_W

def _make_mesh():
    return plsc.VectorSubcoreMesh(core_axis_name="c", subcore_axis_name="s")


def _hist_call(ebi2d):
    @functools.partial(
        pl.kernel,
        out_type=(jax.ShapeDtypeStruct((NEP,), jnp.float32),
                  jax.ShapeDtypeStruct((NEP,), jnp.float32)),
        mesh=_make_mesh(),
        scratch_types=[
            pltpu.VMEM_SHARED((NEP,), jnp.float32),
            pltpu.VMEM((DMA_ROWS, IDX_COLS), jnp.int32),
            pltpu.VMEM((IDX_COLS,), jnp.float32),
            pltpu.VMEM((ZB,), jnp.float32),
        ],
    )
    def hist_kernel(ebi_hbm, hist0_hbm, hist1_hbm, hist_sp, idx_v, ones_v, zbuf):
        c = lax.axis_index("c")
        s = lax.axis_index("s")
        wid = s * NC + c

        def fill_z(i, _):
            zbuf[pl.ds(i * LANES, LANES)] = jnp.zeros((LANES,), jnp.float32)
            return 0

        lax.fori_loop(0, ZB // LANES, fill_z, 0)

        def fill_o(i, _):
            ones_v[pl.ds(i * LANES, LANES)] = jnp.ones((LANES,), jnp.float32)
            return 0

        lax.fori_loop(0, IDX_COLS // LANES, fill_o, 0)

        base_sp = s * SP_SLICE

        def zero_sp(i, _):
            pltpu.sync_copy(zbuf, hist_sp.at[pl.ds(base_sp + i * ZB, ZB)])
            return 0

        lax.fori_loop(0, SP_SLICE // ZB, zero_sp, 0)
        plsc.subcore_barrier()

        row0 = wid * ROWS_PER_W

        def step(it, _):
            pltpu.sync_copy(ebi_hbm.at[pl.ds(row0 + it * DMA_ROWS, DMA_ROWS)], idx_v)
            for j in range(DMA_ROWS):
                pltpu.sync_copy(ones_v, hist_sp.at[idx_v.at[j]], add=True)
            return 0

        lax.fori_loop(0, N_DMA, step, 0)
        plsc.subcore_barrier()

        @pl.when(c == 0)
        def _():
            pltpu.sync_copy(hist_sp.at[pl.ds(base_sp, SP_SLICE)],
                            hist0_hbm.at[pl.ds(base_sp, SP_SLICE)])

        @pl.when(c == 1)
        def _():
            pltpu.sync_copy(hist_sp.at[pl.ds(base_sp, SP_SLICE)],
                            hist1_hbm.at[pl.ds(base_sp, SP_SLICE)])

    return hist_kernel(ebi2d)


def _wsum_call(hist0, hist1, w0f, w1f, w2f, w3f):
    @functools.partial(
        pl.kernel,
        out_type=jax.ShapeDtypeStruct((NW * 6 * LANES,), jnp.float32),
        mesh=_make_mesh(),
        scratch_types=[
            pltpu.VMEM((CHUNK,), jnp.float32),
            pltpu.VMEM((CHUNK,), jnp.float32),
            pltpu.VMEM((CHUNK,), jnp.float32),
            pltpu.VMEM((CHUNK,), jnp.float32),
            pltpu.VMEM((CHUNK // 3,), jnp.float32),
            pltpu.VMEM((CHUNK // 3,), jnp.float32),
            pltpu.VMEM((6 * LANES,), jnp.float32),
        ],
        compiler_params=pltpu.CompilerParams(needs_layout_passes=False),
    )
    def wsum_kernel(h0_hbm, h1_hbm, w0_hbm, w1_hbm, w2_hbm, w3_hbm, out_hbm,
                    wb0, wb1, wb2, wb3, hb0, hb1, ob):
        c = lax.axis_index("c")
        s = lax.axis_index("s")
        wid = s * NC + c
        fb = wid * PER_W
        hbase = wid * (PER_W // 3)

        lane = lax.iota(jnp.int32, LANES)
        lane3 = lane * 3

        def load_chunk(fo, ho, nf):
            pltpu.sync_copy(w0_hbm.at[pl.ds(fo, nf)], wb0.at[pl.ds(0, nf)])
            pltpu.sync_copy(w1_hbm.at[pl.ds(fo, nf)], wb1.at[pl.ds(0, nf)])
            pltpu.sync_copy(w2_hbm.at[pl.ds(fo, nf)], wb2.at[pl.ds(0, nf)])
            pltpu.sync_copy(w3_hbm.at[pl.ds(fo, nf)], wb3.at[pl.ds(0, nf)])
            pltpu.sync_copy(h0_hbm.at[pl.ds(ho, nf // 3)], hb0.at[pl.ds(0, nf // 3)])
            pltpu.sync_copy(h1_hbm.at[pl.ds(ho, nf // 3)], hb1.at[pl.ds(0, nf // 3)])

        def rowgroup_body(gate):
            def body(t, accs):
                a = list(accs)
                h = hb0[pl.ds(t * LANES, LANES)] + hb1[pl.ds(t * LANES, LANES)]
                if gate is not None:
                    h = h * gate
                i0 = lane3 + t * 48
                for cc in range(3):
                    ic = i0 + cc
                    w0v = plsc.load_gather(wb0, [ic])
                    w2v = plsc.load_gather(wb2, [ic])
                    a[cc] = a[cc] + h * (w0v + w2v)
                    w1v = plsc.load_gather(wb1, [ic])
                    w3v = plsc.load_gather(wb3, [ic])
                    a[3 + cc] = a[3 + cc] + h * (w1v + w3v)
                return tuple(a)
            return body

        zero16 = jnp.zeros((LANES,), jnp.float32)
        accs = (zero16,) * 6

        def chunk_body(k, accs):
            load_chunk(fb + k * CHUNK, hbase + k * (CHUNK // 3), CHUNK)
            return lax.fori_loop(0, CHUNK // 48, rowgroup_body(None), accs)

        accs = lax.fori_loop(0, NCH, chunk_body, accs)

        load_chunk(fb + NCH * CHUNK, hbase + NCH * (CHUNK // 3), TAIL)
        accs = lax.fori_loop(0, TAIL // 48, rowgroup_body(None), accs)

        load_chunk(FLAT - EXTRA, (FLAT - EXTRA) // 3, EXTRA)
        gate = jnp.where(wid == NW - 1, 1.0, 0.0).astype(jnp.float32)
        accs = lax.fori_loop(0, EXTRA // 48, rowgroup_body(gate), accs)

        for i in range(6):
            ob[pl.ds(i * LANES, LANES)] = accs[i]
        pltpu.sync_copy(ob, out_hbm.at[pl.ds(wid * 6 * LANES, 6 * LANES)])

    return wsum_kernel(hist0, hist1, w0f, w1f, w2f, w3f)


def kernel(eb_input, eb_offset, W0, W1, W2, W3):
    del eb_offset
    ebi2d = eb_input.reshape(IDX_ROWS_TOTAL, IDX_COLS)
    hist0, hist1 = _hist_call(ebi2d)
    partials = _wsum_call(hist0, hist1, W0.reshape(-1), W1.reshape(-1),
                          W2.reshape(-1), W3.reshape(-1))
    return jnp.sum(partials.reshape(NW, 6, LANES), axis=(0, 2))


# speedup vs baseline: 332.9265x; 32.8869x over previous
import functools

import numpy as np
import jax
import jax.numpy as jnp
from jax import lax
from jax.experimental import pallas as pl
from jax.experimental.pallas import tpu as pltpu
from jax.experimental.pallas import tpu_sc as plsc

NUM_EMB_ROWS = 1_000_000
NUM_IDX = 819_200
NC = 2
NS = 16
NW = NC * NS
LANES = 16

NEP = 1 << 20
IDX_COLS = 128
IDX_ROWS_TOTAL = NUM_IDX // IDX_COLS
ROWS_PER_W = IDX_ROWS_TOTAL // NW
DMA_ROWS = 8
N_DMA = ROWS_PER_W // DMA_ROWS
SP_SLICE = NEP // NS
ZB = 8192

ROWS_W = 31_248
RCHUNK = 2_608
NRCH = 11
RTAIL = ROWS_W - NRCH * RCHUNK
REXTRA = NUM_EMB_ROWS - NW * ROWS_W

def _make_mesh():
    return plsc.VectorSubcoreMesh(core_axis_name="c", subcore_axis_name="s")


def _hist_call(ebi2d):
    @functools.partial(
        pl.kernel,
        out_type=(jax.ShapeDtypeStruct((NEP,), jnp.float32),
                  jax.ShapeDtypeStruct((NEP,), jnp.float32)),
        mesh=_make_mesh(),
        scratch_types=[
            pltpu.VMEM_SHARED((NEP,), jnp.float32),
            pltpu.VMEM((DMA_ROWS, IDX_COLS), jnp.int32),
            pltpu.VMEM((IDX_COLS,), jnp.float32),
            pltpu.VMEM((ZB,), jnp.float32),
        ],
    )
    def hist_kernel(ebi_hbm, hist0_hbm, hist1_hbm, hist_sp, idx_v, ones_v, zbuf):
        c = lax.axis_index("c")
        s = lax.axis_index("s")
        wid = s * NC + c

        def fill_z(i, _):
            zbuf[pl.ds(i * LANES, LANES)] = jnp.zeros((LANES,), jnp.float32)
            return 0

        lax.fori_loop(0, ZB // LANES, fill_z, 0)

        def fill_o(i, _):
            ones_v[pl.ds(i * LANES, LANES)] = jnp.ones((LANES,), jnp.float32)
            return 0

        lax.fori_loop(0, IDX_COLS // LANES, fill_o, 0)

        base_sp = s * SP_SLICE

        def zero_sp(i, _):
            pltpu.sync_copy(zbuf, hist_sp.at[pl.ds(base_sp + i * ZB, ZB)])
            return 0

        lax.fori_loop(0, SP_SLICE // ZB, zero_sp, 0)
        plsc.subcore_barrier()

        row0 = wid * ROWS_PER_W

        def step(it, _):
            pltpu.sync_copy(ebi_hbm.at[pl.ds(row0 + it * DMA_ROWS, DMA_ROWS)], idx_v)
            for j in range(DMA_ROWS):
                pltpu.sync_copy(ones_v, hist_sp.at[idx_v.at[j]], add=True)
            return 0

        lax.fori_loop(0, N_DMA, step, 0)
        plsc.subcore_barrier()

        @pl.when(c == 0)
        def _():
            pltpu.sync_copy(hist_sp.at[pl.ds(base_sp, SP_SLICE)],
                            hist0_hbm.at[pl.ds(base_sp, SP_SLICE)])

        @pl.when(c == 1)
        def _():
            pltpu.sync_copy(hist_sp.at[pl.ds(base_sp, SP_SLICE)],
                            hist1_hbm.at[pl.ds(base_sp, SP_SLICE)])

    return hist_kernel(ebi2d)


def _wsum_call(hist0, hist1, wcols):
    @functools.partial(
        pl.kernel,
        out_type=jax.ShapeDtypeStruct((NW * 6 * LANES,), jnp.float32),
        mesh=_make_mesh(),
        scratch_types=[
            [pltpu.VMEM((RCHUNK,), jnp.float32) for _ in range(12)],
            pltpu.VMEM((RCHUNK,), jnp.float32),
            pltpu.VMEM((RCHUNK,), jnp.float32),
            pltpu.VMEM((6 * LANES,), jnp.float32),
        ],
        compiler_params=pltpu.CompilerParams(needs_layout_passes=False),
    )
    def wsum_kernel(h0_hbm, h1_hbm, *rest):
        wc_hbm = rest[:12]
        out_hbm = rest[12]
        wcb = rest[13]
        hb0, hb1, ob = rest[14], rest[15], rest[16]
        c = lax.axis_index("c")
        s = lax.axis_index("s")
        wid = s * NC + c
        rbase = wid * ROWS_W

        def load_chunk(ro, nr):
            for i in range(12):
                pltpu.sync_copy(wc_hbm[i].at[pl.ds(ro, nr)], wcb[i].at[pl.ds(0, nr)])
            pltpu.sync_copy(h0_hbm.at[pl.ds(ro, nr)], hb0.at[pl.ds(0, nr)])
            pltpu.sync_copy(h1_hbm.at[pl.ds(ro, nr)], hb1.at[pl.ds(0, nr)])

        def rowgroup_body(gate):
            def body(t, accs):
                a = list(accs)
                sl = pl.ds(t * LANES, LANES)
                h = hb0[sl] + hb1[sl]
                if gate is not None:
                    h = h * gate
                for cc in range(3):
                    a[cc] = a[cc] + h * (wcb[0 + cc][sl] + wcb[6 + cc][sl])
                    a[3 + cc] = a[3 + cc] + h * (wcb[3 + cc][sl] + wcb[9 + cc][sl])
                return tuple(a)
            return body

        zero16 = jnp.zeros((LANES,), jnp.float32)
        accs = (zero16,) * 6

        def chunk_body(k, accs):
            load_chunk(rbase + k * RCHUNK, RCHUNK)
            return lax.fori_loop(0, RCHUNK // LANES, rowgroup_body(None), accs)

        accs = lax.fori_loop(0, NRCH, chunk_body, accs)

        load_chunk(rbase + NRCH * RCHUNK, RTAIL)
        accs = lax.fori_loop(0, RTAIL // LANES, rowgroup_body(None), accs)

        load_chunk(NUM_EMB_ROWS - REXTRA, REXTRA)
        gate = jnp.where(wid == NW - 1, 1.0, 0.0).astype(jnp.float32)
        accs = lax.fori_loop(0, REXTRA // LANES, rowgroup_body(gate), accs)

        for i in range(6):
            ob[pl.ds(i * LANES, LANES)] = accs[i]
        pltpu.sync_copy(ob, out_hbm.at[pl.ds(wid * 6 * LANES, 6 * LANES)])

    return wsum_kernel(hist0, hist1, *wcols)


def kernel(eb_input, eb_offset, W0, W1, W2, W3):
    del eb_offset
    ebi2d = eb_input.reshape(IDX_ROWS_TOTAL, IDX_COLS)
    hist0, hist1 = _hist_call(ebi2d)
    wcols = [W[:, cc] for W in (W0, W1, W2, W3) for cc in range(3)]
    partials = _wsum_call(hist0, hist1, wcols)
    return jnp.sum(partials.reshape(NW, 6, LANES), axis=(0, 2))


# speedup vs baseline: 473.0108x; 1.4208x over previous
import functools

import numpy as np
import jax
import jax.numpy as jnp
from jax import lax
from jax.experimental import pallas as pl
from jax.experimental.pallas import tpu as pltpu
from jax.experimental.pallas import tpu_sc as plsc

NUM_EMB_ROWS = 1_000_000
NUM_IDX = 819_200
NC = 2
NS = 16
NW = NC * NS
LANES = 16

NEP = 1 << 20
IDX_COLS = 128
IDX_ROWS_TOTAL = NUM_IDX // IDX_COLS
ROWS_PER_W = IDX_ROWS_TOTAL // NW
SCAT_AHEAD = 16
SP_SLICE = NEP // NS
ZB = 8192

ROWS_W = 31_248
RCHUNK = 2_608
NRCH = 11
RTAIL = ROWS_W - NRCH * RCHUNK
REXTRA = NUM_EMB_ROWS - NW * ROWS_W

def _make_mesh():
    return plsc.VectorSubcoreMesh(core_axis_name="c", subcore_axis_name="s")


def _hist_call(ebi2d):
    @functools.partial(
        pl.kernel,
        out_type=(jax.ShapeDtypeStruct((NEP,), jnp.float32),
                  jax.ShapeDtypeStruct((NEP,), jnp.float32)),
        mesh=_make_mesh(),
        scratch_types=[
            pltpu.VMEM_SHARED((NEP,), jnp.float32),
            pltpu.VMEM((ROWS_PER_W, IDX_COLS), jnp.int32),
            pltpu.VMEM((IDX_COLS,), jnp.float32),
            pltpu.VMEM((ZB,), jnp.float32),
            pltpu.SemaphoreType.DMA,
        ],
    )
    def hist_kernel(ebi_hbm, hist0_hbm, hist1_hbm, hist_sp, idx_v, ones_v, zbuf, sem):
        c = lax.axis_index("c")
        s = lax.axis_index("s")
        wid = s * NC + c

        def fill_z(i, _):
            zbuf[pl.ds(i * LANES, LANES)] = jnp.zeros((LANES,), jnp.float32)
            return 0

        lax.fori_loop(0, ZB // LANES, fill_z, 0)

        def fill_o(i, _):
            ones_v[pl.ds(i * LANES, LANES)] = jnp.ones((LANES,), jnp.float32)
            return 0

        lax.fori_loop(0, IDX_COLS // LANES, fill_o, 0)

        base_sp = s * SP_SLICE

        def zero_sp(i, _):
            pltpu.sync_copy(zbuf, hist_sp.at[pl.ds(base_sp + i * ZB, ZB)])
            return 0

        lax.fori_loop(0, SP_SLICE // ZB, zero_sp, 0)
        plsc.subcore_barrier()

        row0 = wid * ROWS_PER_W
        pltpu.sync_copy(ebi_hbm.at[pl.ds(row0, ROWS_PER_W)], idx_v)
        scat = []
        for g in range(ROWS_PER_W):
            scat.append(pltpu.async_copy(
                ones_v, hist_sp.at[idx_v.at[g]], sem, add=True))
            if g >= SCAT_AHEAD:
                scat[g - SCAT_AHEAD].wait()
        for g in range(ROWS_PER_W - SCAT_AHEAD, ROWS_PER_W):
            scat[g].wait()
        plsc.subcore_barrier()

        @pl.when(c == 0)
        def _():
            pltpu.sync_copy(hist_sp.at[pl.ds(base_sp, SP_SLICE)],
                            hist0_hbm.at[pl.ds(base_sp, SP_SLICE)])

        @pl.when(c == 1)
        def _():
            pltpu.sync_copy(hist_sp.at[pl.ds(base_sp, SP_SLICE)],
                            hist1_hbm.at[pl.ds(base_sp, SP_SLICE)])

    return hist_kernel(ebi2d)


def _wsum_call(hist0, hist1, wcols):
    @functools.partial(
        pl.kernel,
        out_type=jax.ShapeDtypeStruct((NW * 6 * LANES,), jnp.float32),
        mesh=_make_mesh(),
        scratch_types=[
            [[pltpu.VMEM((RCHUNK,), jnp.float32) for _ in range(14)]
             for _ in range(2)],
            pltpu.VMEM((6 * LANES,), jnp.float32),
            [pltpu.SemaphoreType.DMA for _ in range(2)],
        ],
        compiler_params=pltpu.CompilerParams(needs_layout_passes=False),
    )
    def wsum_kernel(h0_hbm, h1_hbm, *rest):
        wc_hbm = rest[:12]
        out_hbm = rest[12]
        bufs = rest[13]
        ob = rest[14]
        sems = rest[15]
        c = lax.axis_index("c")
        s = lax.axis_index("s")
        wid = s * NC + c
        rbase = wid * ROWS_W

        def fire(sidx, ro, nr):
            ds = []
            for i in range(12):
                ds.append(pltpu.async_copy(
                    wc_hbm[i].at[pl.ds(ro, nr)],
                    bufs[sidx][i].at[pl.ds(0, nr)], sems[sidx]))
            ds.append(pltpu.async_copy(
                h0_hbm.at[pl.ds(ro, nr)], bufs[sidx][12].at[pl.ds(0, nr)],
                sems[sidx]))
            ds.append(pltpu.async_copy(
                h1_hbm.at[pl.ds(ro, nr)], bufs[sidx][13].at[pl.ds(0, nr)],
                sems[sidx]))
            return ds

        def rowgroup_body(sidx, gate):
            wcb = bufs[sidx]

            def body(t, accs):
                a = list(accs)
                sl = pl.ds(t * LANES, LANES)
                h = wcb[12][sl] + wcb[13][sl]
                if gate is not None:
                    h = h * gate
                for cc in range(3):
                    a[cc] = a[cc] + h * (wcb[0 + cc][sl] + wcb[6 + cc][sl])
                    a[3 + cc] = a[3 + cc] + h * (wcb[3 + cc][sl] + wcb[9 + cc][sl])
                return tuple(a)
            return body

        zero16 = jnp.zeros((LANES,), jnp.float32)
        accs = (zero16,) * 6

        gate = jnp.where(wid == NW - 1, 1.0, 0.0).astype(jnp.float32)
        stages = [(rbase + k * RCHUNK, RCHUNK, None) for k in range(NRCH)]
        stages.append((rbase + NRCH * RCHUNK, RTAIL, None))
        stages.append((NUM_EMB_ROWS - REXTRA, REXTRA, gate))

        descs = fire(0, stages[0][0], stages[0][1])
        for i, (ro, nr, g) in enumerate(stages):
            sidx = i % 2
            nxt = None
            if i + 1 < len(stages):
                nxt = fire(1 - sidx, stages[i + 1][0], stages[i + 1][1])
            for d in descs:
                d.wait()
            accs = lax.fori_loop(0, nr // LANES, rowgroup_body(sidx, g), accs)
            descs = nxt

        for i in range(6):
            ob[pl.ds(i * LANES, LANES)] = accs[i]
        pltpu.sync_copy(ob, out_hbm.at[pl.ds(wid * 6 * LANES, 6 * LANES)])

    return wsum_kernel(hist0, hist1, *wcols)


def kernel(eb_input, eb_offset, W0, W1, W2, W3):
    del eb_offset
    ebi2d = eb_input.reshape(IDX_ROWS_TOTAL, IDX_COLS)
    hist0, hist1 = _hist_call(ebi2d)
    wcols = [W[:, cc] for W in (W0, W1, W2, W3) for cc in range(3)]
    partials = _wsum_call(hist0, hist1, wcols)
    return jnp.sum(partials.reshape(NW, 6, LANES), axis=(0, 2))
